# baseline (speedup 1.0000x reference)
import functools

import jax
import jax.numpy as jnp
from jax import lax
from jax.experimental import pallas as pl
from jax.experimental.pallas import tpu as pltpu
from jax.experimental.pallas import tpu_sc as plsc

D = 64
L = 16
NC, NS = 2, 16
NW = NC * NS
B = 4096
SEQ = 200
SEQ_PAD = 208
ROWS_PER_W = B // NW
TOK_W = ROWS_PER_W * SEQ_PAD
CHUNKS = SEQ_PAD // L
RPU = 2
UNIT = RPU * SEQ_PAD
UNITS = ROWS_PER_W // RPU
RING = 4
VOCAB = 1000000
PROJ_BLK = 32768
PROJ_GRID = -(-VOCAB // PROJ_BLK)
VOCAB_PAD = PROJ_GRID * PROJ_BLK

_SLICES = [(0, 128), (128, 128), (256, 128), (384, 32)]

_mesh = plsc.VectorSubcoreMesh(
    core_axis_name="c", subcore_axis_name="s", num_cores=NC, num_subcores=NS
)


def _tc_project(tab_ref, w_ref, p_ref):
    t = tab_ref[...]
    w = w_ref[...]
    p_ref[...] = lax.dot_general(
        w, t, (((1,), (1,)), ((), ())),
        preferred_element_type=jnp.float32)


@functools.partial(
    pl.kernel,
    out_type=jax.ShapeDtypeStruct((B, 3 * L), jnp.float32),
    mesh=_mesh,
    compiler_params=pltpu.CompilerParams(use_tc_tiling_on_sc=False),
    scratch_types=[
        pltpu.VMEM((TOK_W,), jnp.int32),
        pltpu.VMEM((RING, UNIT), jnp.float32),
        pltpu.VMEM((RING, UNIT), jnp.float32),
        pltpu.VMEM((ROWS_PER_W, 3 * L), jnp.float32),
        pltpu.SemaphoreType.DMA,
        pltpu.SemaphoreType.DMA,
        pltpu.SemaphoreType.DMA,
        pltpu.SemaphoreType.DMA,
    ],
)
def _sc_pool(ids_hbm, p0_hbm, p1_hbm, parts_hbm,
             ids_v, g0_v, g1_v, parts_v, sem0, sem1, sem2, sem3):
    sems = (sem0, sem1, sem2, sem3)
    wid = lax.axis_index("s") * NC + lax.axis_index("c")
    base = wid * ROWS_PER_W

    pltpu.sync_copy(ids_hbm.at[pl.ds(wid * TOK_W, TOK_W)], ids_v)

    def start_unit(u, buf):
        off = u * UNIT
        sem = sems[buf]
        for s, n in _SLICES:
            idx = ids_v.at[pl.ds(off + s, n)]
            pltpu.async_copy(p0_hbm.at[idx], g0_v.at[buf, pl.ds(s, n)], sem)
            pltpu.async_copy(p1_hbm.at[idx], g1_v.at[buf, pl.ds(s, n)], sem)

    def wait_unit(buf):
        sem = sems[buf]
        pltpu.make_async_copy(
            p0_hbm.at[pl.ds(0, UNIT)], g0_v.at[buf], sem).wait()
        pltpu.make_async_copy(
            p1_hbm.at[pl.ds(0, UNIT)], g1_v.at[buf], sem).wait()

    for u in range(RING - 1):
        start_unit(u, u)

    def do_unit(u, buf):
        @pl.when(u + RING - 1 < UNITS)
        def _():
            start_unit(u + RING - 1, (buf + RING - 1) % RING)

        wait_unit(buf)

        for r in range(RPU):
            row = u * RPU + r
            z = jnp.zeros((L,), jnp.float32)
            a0e, a0o, a1e, a1o, cnt = z, z, z, z, z
            for c in range(CHUNKS):
                lane = r * SEQ_PAD + c * L
                if c & 1:
                    a0o = a0o + g0_v[buf, pl.ds(lane, L)]
                    a1o = a1o + g1_v[buf, pl.ds(lane, L)]
                else:
                    a0e = a0e + g0_v[buf, pl.ds(lane, L)]
                    a1e = a1e + g1_v[buf, pl.ds(lane, L)]
                ids16 = ids_v[pl.ds(u * UNIT + lane, L)]
                cnt = cnt + jnp.where(ids16 != 0, 1.0, 0.0)
            parts_v[row, pl.ds(0, L)] = a0e + a0o
            parts_v[row, pl.ds(L, L)] = a1e + a1o
            parts_v[row, pl.ds(2 * L, L)] = cnt

    @pl.loop(0, UNITS, step=RING)
    def _(u0):
        for k in range(RING):
            do_unit(u0 + k, k)

    pltpu.sync_copy(parts_v, parts_hbm.at[pl.ds(base, ROWS_PER_W)])


def _tc_head(parts_ref, b_ref, out_ref):
    p = parts_ref[...]
    c0 = jnp.sum(p[:, 0:L], axis=1, keepdims=True)
    c1 = jnp.sum(p[:, L:2 * L], axis=1, keepdims=True)
    cnt = jnp.sum(p[:, 2 * L:3 * L], axis=1, keepdims=True)
    se = jnp.concatenate([c0, c1], axis=1) / (cnt + 1e-8)
    out_ref[...] = se + b_ref[...]


def kernel(input_ids, table, W, b):
    ids = input_ids.astype(jnp.int32)
    ids_flat = jnp.pad(ids, ((0, 0), (0, SEQ_PAD - SEQ))).reshape(-1)
    p01 = pl.pallas_call(
        _tc_project,
        grid=(PROJ_GRID,),
        in_specs=[
            pl.BlockSpec((PROJ_BLK, D), lambda i: (i, 0)),
            pl.BlockSpec((2, D), lambda i: (0, 0)),
        ],
        out_specs=pl.BlockSpec((2, PROJ_BLK), lambda i: (0, i)),
        out_shape=jax.ShapeDtypeStruct((2, VOCAB_PAD), jnp.float32),
    )(table, W)
    parts = _sc_pool(ids_flat, p01[0], p01[1])
    logits = pl.pallas_call(
        _tc_head,
        out_shape=jax.ShapeDtypeStruct((B, W.shape[0]), jnp.float32),
    )(parts, b.reshape(1, -1))
    return logits
